# Initial kernel scaffold; baseline (speedup 1.0000x reference)
import jax
import jax.numpy as jnp
from jax.experimental import pallas as pl


def kernel(x, edge_index, W_e, b_e, W_ih, b_ih, W_hh, b_hh, W_gc, b_gc, W_pool, b_pool, W_out, b_out):
    raise NotImplementedError("write your pallas kernel here")


# speedup vs baseline: 3.8453x; 3.8453x over previous
import functools

import jax
import jax.numpy as jnp
from jax import lax
from jax.experimental import pallas as pl
from jax.experimental.pallas import tpu as pltpu
from jax.experimental.pallas import tpu_sc as plsc

_N = 10000
_E = 320000
_H = 128
_C = 3
_T = 5

_NC = 2
_NS = 16
_NW = _NC * _NS
_EPW = _E // _NW
_CH = 80
_NCH = _EPW // _CH
_NP = 10240
_RPT = _NP // _NS

_f32 = jnp.float32


@functools.lru_cache(maxsize=None)
def _make_sc_kernels():
  _mesh = plsc.VectorSubcoreMesh(core_axis_name="c", subcore_axis_name="s",
                                 num_cores=_NC, num_subcores=_NS)

  @functools.partial(
      pl.kernel,
      out_type=jax.ShapeDtypeStruct((_NC * _NP, _H), _f32),
      mesh=_mesh,
      scratch_types=[
          pltpu.VMEM_SHARED((_NP, _H), _f32),
          pltpu.VMEM((_CH,), jnp.int32),
          pltpu.VMEM((_CH,), jnp.int32),
          pltpu.VMEM((_CH, _H), _f32),
          pltpu.SemaphoreType.DMA,
      ],
  )
  def _sc_spmm(mh, src, dst, zrows, out, acc, sidx, didx, rows, sem):
      c = lax.axis_index("c")
      s = lax.axis_index("s")
      wid = s * _NC + c
      pltpu.sync_copy(zrows, acc.at[pl.ds(s * _RPT, _RPT)])
      plsc.subcore_barrier()
      ebase = wid * _EPW

      def body(i, carry):
          base = pl.multiple_of(ebase + i * _CH, 8)
          pltpu.sync_copy(src.at[pl.ds(base, _CH)], sidx)
          pltpu.sync_copy(dst.at[pl.ds(base, _CH)], didx)
          pltpu.async_copy(mh.at[sidx], rows, sem).wait()
          pltpu.sync_copy(rows, acc.at[didx], add=True)
          return carry

      lax.fori_loop(0, _NCH, body, 0)
      plsc.subcore_barrier()
      pltpu.sync_copy(acc.at[pl.ds(s * _RPT, _RPT)],
                      out.at[pl.ds(c * _NP + s * _RPT, _RPT)])


  @functools.partial(
      pl.kernel,
      out_type=[jax.ShapeDtypeStruct((_NC * _NP, _H), _f32),
                jax.ShapeDtypeStruct((_NC * _NP, _H), _f32)],
      mesh=_mesh,
      scratch_types=[
          pltpu.VMEM_SHARED((_NP, _H), _f32),
          pltpu.VMEM((_CH,), jnp.int32),
          pltpu.VMEM((_CH, _H), _f32),
      ],
  )
  def _sc_deg(src, dst, ones_rows, zrows, dego_out, degi_out,
              tab, sidx, ones_v):
      c = lax.axis_index("c")
      s = lax.axis_index("s")
      wid = s * _NC + c
      ebase = wid * _EPW
      pltpu.sync_copy(ones_rows, ones_v)

      def count(idx_hbm, out_hbm):
          pltpu.sync_copy(zrows, tab.at[pl.ds(s * _RPT, _RPT)])
          plsc.subcore_barrier()

          def body(i, carry):
              base = pl.multiple_of(ebase + i * _CH, 8)
              pltpu.sync_copy(idx_hbm.at[pl.ds(base, _CH)], sidx)
              pltpu.sync_copy(ones_v, tab.at[sidx], add=True)
              return carry

          lax.fori_loop(0, _NCH, body, 0)
          plsc.subcore_barrier()
          pltpu.sync_copy(tab.at[pl.ds(s * _RPT, _RPT)],
                          out_hbm.at[pl.ds(c * _NP + s * _RPT, _RPT)])

      count(src, dego_out)
      plsc.subcore_barrier()
      count(dst, degi_out)

  return _sc_spmm, _sc_deg


def _rowmajor_matmul(a, w_t, b):
    return lax.dot_general(a, w_t, (((1,), (1,)), ((), ())),
                           preferred_element_type=_f32) + b


def _tc_init_body(x_ref, we_ref, be_ref, h_ref, mh_ref):
    h = x_ref[...] / jnp.sqrt(jnp.float32(_H))
    h_ref[...] = h
    mh_ref[...] = _rowmajor_matmul(h, we_ref[...], be_ref[...])


_tc_init = pl.pallas_call(
    _tc_init_body,
    out_shape=[jax.ShapeDtypeStruct((_N, _H), _f32),
               jax.ShapeDtypeStruct((_N, _H), _f32)],
)


def _tc_gru_body(ap_ref, h_ref, wih_ref, bih_ref, whh_ref, bhh_ref,
                 we_ref, be_ref, hn_ref, mh_ref):
    a = ap_ref[0] + ap_ref[1]
    h = h_ref[...]
    gi = _rowmajor_matmul(a, wih_ref[...], bih_ref[...])
    gh = _rowmajor_matmul(h, whh_ref[...], bhh_ref[...])
    r = jax.nn.sigmoid(gi[:, :_H] + gh[:, :_H])
    z = jax.nn.sigmoid(gi[:, _H:2 * _H] + gh[:, _H:2 * _H])
    n = jnp.tanh(gi[:, 2 * _H:] + r * gh[:, 2 * _H:])
    hn = (1.0 - z) * n + z * h
    hn_ref[...] = hn
    mh_ref[...] = _rowmajor_matmul(hn, we_ref[...], be_ref[...])


_tc_gru = pl.pallas_call(
    _tc_gru_body,
    out_shape=[jax.ShapeDtypeStruct((_N, _H), _f32),
               jax.ShapeDtypeStruct((_N, _H), _f32)],
)


def _tc_hs_body(h_ref, dego_ref, hs_ref):
    deg = dego_ref[0, :, 0:1] + dego_ref[1, :, 0:1]
    hs_ref[...] = h_ref[...] * lax.rsqrt(jnp.maximum(deg, 1.0))


_tc_hs = pl.pallas_call(
    _tc_hs_body,
    out_shape=jax.ShapeDtypeStruct((_N, _H), _f32),
)


def _tc_fin_body(aggp_ref, degi_ref, wgc_ref, bgc_ref, wpool_ref,
                 bpool_ref, wout_ref, bout_ref, colors_ref, sat_ref):
    deg = degi_ref[0, :, 0:1] + degi_ref[1, :, 0:1]
    agg = (aggp_ref[0] + aggp_ref[1]) * lax.rsqrt(jnp.maximum(deg, 1.0))
    gc = jnp.dot(agg, wgc_ref[...], preferred_element_type=_f32) + bgc_ref[...]
    colors = jax.nn.softmax(gc, axis=1)
    colors_ref[...] = colors
    pool = jnp.dot(colors, wpool_ref[...],
                   preferred_element_type=_f32) + bpool_ref[...]
    gate = jax.nn.softmax(pool, axis=0)
    readout = jnp.sum(gate * colors, axis=0, keepdims=True)
    sat_ref[...] = jax.nn.sigmoid(
        jnp.dot(readout, wout_ref[...], preferred_element_type=_f32)
        + bout_ref[...])


_tc_fin = pl.pallas_call(
    _tc_fin_body,
    out_shape=[jax.ShapeDtypeStruct((_N, _C), _f32),
               jax.ShapeDtypeStruct((1, 1), _f32)],
)


def kernel(x, edge_index, W_e, b_e, W_ih, b_ih, W_hh, b_hh,
           W_gc, b_gc, W_pool, b_pool, W_out, b_out):
    src = edge_index[0]
    dst = edge_index[1]
    zrows = jnp.zeros((_RPT, _H), _f32)
    ones_rows = jnp.ones((_CH, _H), _f32)
    be2 = b_e.reshape(1, _H)
    bih2 = b_ih.reshape(1, 3 * _H)
    bhh2 = b_hh.reshape(1, 3 * _H)
    bgc2 = b_gc.reshape(1, _C)
    bpool2 = b_pool.reshape(1, 1)
    bout2 = b_out.reshape(1, 1)

    sc_spmm, sc_deg = _make_sc_kernels()
    h, mh = _tc_init(x, W_e, be2)
    dego, degi = sc_deg(src, dst, ones_rows, zrows)
    dego = dego.reshape(_NC, _NP, _H)[:, :_N, :1]
    degi = degi.reshape(_NC, _NP, _H)[:, :_N, :1]
    for _ in range(_T):
        ap = sc_spmm(mh, src, dst, zrows).reshape(_NC, _NP, _H)[:, :_N, :]
        h, mh = _tc_gru(ap, h, W_ih, bih2, W_hh, bhh2, W_e, be2)
    hs = _tc_hs(h, dego)
    aggp = sc_spmm(hs, src, dst, zrows).reshape(_NC, _NP, _H)[:, :_N, :]
    colors, sat = _tc_fin(aggp, degi, W_gc, bgc2,
                          W_pool, bpool2, W_out, bout2)
    return colors, sat.reshape(())
